# baseline (speedup 1.0000x reference)
import functools

import jax
import jax.numpy as jnp
from jax import lax
from jax.experimental import pallas as pl
from jax.experimental.pallas import tpu as pltpu
from jax.experimental.pallas import tpu_sc as plsc

NN = 10000
HH = 128
NP = 10240
EE = 320000
NC = 2
NS = 16
NW = NC * NS
KE = 128
CH = 80
CHA = 80
CHB = 80
EP = NW * CH * KE
RPT = NP // NS
RB = 1024
GG = NP // RB

_mesh = plsc.VectorSubcoreMesh(core_axis_name="c", subcore_axis_name="s")


def _deg_call(dst3, zcol):
    @functools.partial(
        pl.kernel,
        out_type=jax.ShapeDtypeStruct((NC, NP), jnp.float32),
        mesh=_mesh,
        scratch_types=[
            pltpu.VMEM((CH, KE), jnp.int32),
            pltpu.VMEM((KE,), jnp.float32),
            pltpu.VMEM_SHARED((NP,), jnp.float32),
        ],
    )
    def deg_k(dst_hbm, zcol_hbm, out_hbm, dst_v, ones_v, deg_sh):
        cid = lax.axis_index("c")
        sid = lax.axis_index("s")
        wid = sid * NC + cid
        pltpu.sync_copy(dst_hbm.at[wid], dst_v)

        def fill(i, _):
            ones_v[pl.ds(i * 16, 16)] = jnp.ones((16,), jnp.float32)
            return 0

        lax.fori_loop(0, KE // 16, fill, 0)
        pltpu.sync_copy(zcol_hbm, deg_sh.at[pl.ds(sid * RPT, RPT)])
        plsc.subcore_barrier()

        def body(j, _):
            pltpu.sync_copy(ones_v, deg_sh.at[dst_v.at[j]], add=True)
            return 0

        lax.fori_loop(0, CH, body, 0)
        plsc.subcore_barrier()
        pltpu.sync_copy(
            deg_sh.at[pl.ds(sid * RPT, RPT)], out_hbm.at[cid, pl.ds(sid * RPT, RPT)]
        )

    return deg_k(dst3, zcol)


def _scatter_call(u, srcA, dstA, zrows):
    @functools.partial(
        pl.kernel,
        out_type=jax.ShapeDtypeStruct((NC, NP, HH), jnp.float32),
        mesh=_mesh,
        scratch_types=[
            pltpu.VMEM((CH, KE), jnp.int32),
            pltpu.VMEM((CH, KE), jnp.int32),
            pltpu.VMEM((KE, HH), jnp.float32),
            pltpu.VMEM_SHARED((NP, HH), jnp.float32),
        ],
    )
    def scat_k(
        u_hbm, src_hbm, dst_hbm, zr_hbm, out_hbm,
        src_v, dst_v, rows_v, acc_sh,
    ):
        cid = lax.axis_index("c")
        sid = lax.axis_index("s")
        wid = sid * NC + cid
        pltpu.sync_copy(src_hbm.at[wid], src_v)
        pltpu.sync_copy(dst_hbm.at[wid], dst_v)
        pltpu.sync_copy(zr_hbm, acc_sh.at[pl.ds(sid * RPT, RPT)])
        plsc.subcore_barrier()

        def body(j, _):
            pltpu.sync_copy(u_hbm.at[src_v.at[j]], rows_v)
            pltpu.sync_copy(rows_v, acc_sh.at[dst_v.at[j]], add=True)
            return 0

        lax.fori_loop(0, CH, body, 0)
        plsc.subcore_barrier()
        pltpu.sync_copy(
            acc_sh.at[pl.ds(sid * RPT, RPT)],
            out_hbm.at[cid, pl.ds(sid * RPT, RPT)],
        )

    return scat_k(u, srcA, dstA, zrows)


def _const_spec(shape):
    return pl.BlockSpec(shape, lambda i: tuple(0 for _ in shape))


def _pre_call(x_p, We1, be1, We2, be2, Wg1, degp_t):
    def body(x_ref, w1, b1, w2, b2, wg, dg, u_ref, dv_ref):
        xb = x_ref[...]
        h = jnp.maximum(
            jnp.dot(xb, w1[...], preferred_element_type=jnp.float32) + b1[...], 0.0
        )
        h0 = jnp.dot(h, w2[...], preferred_element_type=jnp.float32) + b2[...]
        deg = jnp.sum(dg[...], axis=1, keepdims=True) + 1.0
        dv = lax.rsqrt(deg)
        u_ref[...] = dv * jnp.dot(h0, wg[...], preferred_element_type=jnp.float32)
        dv_ref[...] = dv

    return pl.pallas_call(
        body,
        grid=(GG,),
        in_specs=[
            pl.BlockSpec((RB, HH), lambda i: (i, 0)),
            _const_spec((HH, HH)),
            _const_spec((1, HH)),
            _const_spec((HH, HH)),
            _const_spec((1, HH)),
            _const_spec((HH, HH)),
            pl.BlockSpec((RB, NC), lambda i: (i, 0)),
        ],
        out_specs=[
            pl.BlockSpec((RB, HH), lambda i: (i, 0)),
            pl.BlockSpec((RB, 1), lambda i: (i, 0)),
        ],
        out_shape=[
            jax.ShapeDtypeStruct((NP, HH), jnp.float32),
            jax.ShapeDtypeStruct((NP, 1), jnp.float32),
        ],
    )(x_p, We1, be1, We2, be2, Wg1, degp_t)


def _mid_call(sp, u, dinv, bg, wgn):
    def body(s_ref, u_ref, dv_ref, b_ref, w_ref, un_ref):
        s = s_ref[0] + s_ref[1]
        dv = dv_ref[...]
        hh = jnp.maximum(dv * (s + u_ref[...]) + b_ref[...], 0.0)
        un_ref[...] = dv * jnp.dot(hh, w_ref[...], preferred_element_type=jnp.float32)

    return pl.pallas_call(
        body,
        grid=(GG,),
        in_specs=[
            pl.BlockSpec((NC, RB, HH), lambda i: (0, i, 0)),
            pl.BlockSpec((RB, HH), lambda i: (i, 0)),
            pl.BlockSpec((RB, 1), lambda i: (i, 0)),
            _const_spec((1, HH)),
            _const_spec((HH, HH)),
        ],
        out_specs=pl.BlockSpec((RB, HH), lambda i: (i, 0)),
        out_shape=jax.ShapeDtypeStruct((NP, HH), jnp.float32),
    )(sp, u, dinv, bg, wgn)


def _post_call(sp, u, dinv, bg3, Wd1, bd1, Wd2, bd2, Wp1, bp1, Wp2, bp2):
    def body(
        s_ref, u_ref, dv_ref, b_ref, wd1, bd1r, wd2, bd2r, wp1, bp1r, wp2, bp2r,
        ge_ref, h_ref, rec_ref, sum_sc, max_sc,
    ):
        i = pl.program_id(0)
        s = s_ref[0] + s_ref[1]
        dv = dv_ref[...]
        hh = jnp.maximum(dv * (s + u_ref[...]) + b_ref[...], 0.0)
        h_ref[...] = hh
        r1 = jnp.maximum(
            jnp.dot(hh, wd1[...], preferred_element_type=jnp.float32) + bd1r[...], 0.0
        )
        rec_ref[...] = (
            jnp.dot(r1, wd2[...], preferred_element_type=jnp.float32) + bd2r[...]
        )
        rows = i * RB + lax.broadcasted_iota(jnp.int32, (RB, 1), 0)
        msk = rows < NN
        ps = jnp.sum(jnp.where(msk, hh, 0.0), axis=0, keepdims=True)
        px = jnp.max(jnp.where(msk, hh, -jnp.inf), axis=0, keepdims=True)

        @pl.when(i == 0)
        def _():
            sum_sc[...] = ps
            max_sc[...] = px

        @pl.when(i > 0)
        def _():
            sum_sc[...] += ps
            max_sc[...] = jnp.maximum(max_sc[...], px)

        @pl.when(i == GG - 1)
        def _():
            g = jnp.concatenate([sum_sc[...] * (1.0 / NN), max_sc[...]], axis=1)
            gh = jnp.maximum(
                jnp.dot(g, wp1[...], preferred_element_type=jnp.float32) + bp1r[...],
                0.0,
            )
            ge_ref[...] = (
                jnp.dot(gh, wp2[...], preferred_element_type=jnp.float32) + bp2r[...]
            )

    return pl.pallas_call(
        body,
        grid=(GG,),
        in_specs=[
            pl.BlockSpec((NC, RB, HH), lambda i: (0, i, 0)),
            pl.BlockSpec((RB, HH), lambda i: (i, 0)),
            pl.BlockSpec((RB, 1), lambda i: (i, 0)),
            _const_spec((1, HH)),
            _const_spec((HH, HH)),
            _const_spec((1, HH)),
            _const_spec((HH, HH)),
            _const_spec((1, HH)),
            _const_spec((2 * HH, HH)),
            _const_spec((1, HH)),
            _const_spec((HH, HH)),
            _const_spec((1, HH)),
        ],
        out_specs=[
            _const_spec((1, HH)),
            pl.BlockSpec((RB, HH), lambda i: (i, 0)),
            pl.BlockSpec((RB, HH), lambda i: (i, 0)),
        ],
        out_shape=[
            jax.ShapeDtypeStruct((1, HH), jnp.float32),
            jax.ShapeDtypeStruct((NP, HH), jnp.float32),
            jax.ShapeDtypeStruct((NP, HH), jnp.float32),
        ],
        scratch_shapes=[
            pltpu.VMEM((1, HH), jnp.float32),
            pltpu.VMEM((1, HH), jnp.float32),
        ],
    )(sp, u, dinv, bg3, Wd1, bd1, Wd2, bd2, Wp1, bp1, Wp2, bp2)


def kernel(
    x, We1, be1, We2, be2, Wg1, bg1, Wg2, bg2, Wg3, bg3,
    Wp1, bp1, Wp2, bp2, Wd1, bd1, Wd2, bd2, edge_index,
):
    f32 = jnp.float32
    src = edge_index[0]
    dst = edge_index[1]
    pad = EP - EE
    src_p = jnp.concatenate([src, jnp.zeros((pad,), src.dtype)])
    pad_dst = NN + (jnp.arange(pad, dtype=dst.dtype) % (NP - NN))
    dst_p = jnp.concatenate([dst, pad_dst])
    dst3 = dst_p.reshape(NW, CH, KE)

    srcA = src_p.reshape(NW, CH, KE)
    dstA = dst_p.reshape(NW, CH, KE)
    x_p = jnp.pad(x, ((0, NP - NN), (0, 0)))
    zcol = jnp.zeros((RPT,), f32)
    zrows = jnp.zeros((RPT, HH), f32)

    degp = _deg_call(dst3, zcol)
    u1, dinv = _pre_call(
        x_p, We1, be1.reshape(1, HH), We2, be2.reshape(1, HH), Wg1, degp.T
    )
    s1 = _scatter_call(u1, srcA, dstA, zrows)
    u2 = _mid_call(s1, u1, dinv, bg1.reshape(1, HH), Wg2)
    s2 = _scatter_call(u2, srcA, dstA, zrows)
    u3 = _mid_call(s2, u2, dinv, bg2.reshape(1, HH), Wg3)
    s3 = _scatter_call(u3, srcA, dstA, zrows)
    ge, h3, rec = _post_call(
        s3, u3, dinv, bg3.reshape(1, HH),
        Wd1, bd1.reshape(1, HH), Wd2, bd2.reshape(1, HH),
        Wp1, bp1.reshape(1, HH), Wp2, bp2.reshape(1, HH),
    )
    return ge, h3[:NN], rec[:NN]


# speedup vs baseline: 1.5031x; 1.5031x over previous
import functools

import jax
import jax.numpy as jnp
from jax import lax
from jax.experimental import pallas as pl
from jax.experimental.pallas import tpu as pltpu
from jax.experimental.pallas import tpu_sc as plsc

NN = 10000
HH = 128
NP = 10240
EE = 320000
NC = 2
NS = 16
NW = NC * NS
KE = 128
CH = 79
EP = NW * CH * KE
RPT = NP // NS
RB = 1024
GG = NP // RB

_mesh = plsc.VectorSubcoreMesh(core_axis_name="c", subcore_axis_name="s")


def _deg_call(dst3, zcol):
    @functools.partial(
        pl.kernel,
        out_type=jax.ShapeDtypeStruct((NC, NP), jnp.float32),
        mesh=_mesh,
        scratch_types=[
            pltpu.VMEM((CH, KE), jnp.int32),
            pltpu.VMEM((KE,), jnp.float32),
            pltpu.VMEM_SHARED((NP,), jnp.float32),
        ],
    )
    def deg_k(dst_hbm, zcol_hbm, out_hbm, dst_v, ones_v, deg_sh):
        cid = lax.axis_index("c")
        sid = lax.axis_index("s")
        wid = sid * NC + cid
        pltpu.sync_copy(dst_hbm.at[wid], dst_v)

        def fill(i, _):
            ones_v[pl.ds(i * 16, 16)] = jnp.ones((16,), jnp.float32)
            return 0

        lax.fori_loop(0, KE // 16, fill, 0)
        pltpu.sync_copy(zcol_hbm, deg_sh.at[pl.ds(sid * RPT, RPT)])
        plsc.subcore_barrier()

        def body(j, _):
            pltpu.sync_copy(ones_v, deg_sh.at[dst_v.at[j]], add=True)
            return 0

        lax.fori_loop(0, CH, body, 0)
        plsc.subcore_barrier()
        pltpu.sync_copy(
            deg_sh.at[pl.ds(sid * RPT, RPT)], out_hbm.at[cid, pl.ds(sid * RPT, RPT)]
        )

    return deg_k(dst3, zcol)


def _scatter_call(u, srcA, dstA, zrows):
    @functools.partial(
        pl.kernel,
        out_type=jax.ShapeDtypeStruct((NC, NP, HH), jnp.float32),
        mesh=_mesh,
        scratch_types=[
            pltpu.VMEM((CH, KE), jnp.int32),
            pltpu.VMEM((CH, KE), jnp.int32),
            pltpu.VMEM((KE, HH), jnp.float32),
            pltpu.VMEM_SHARED((NP, HH), jnp.float32),
        ],
    )
    def scat_k(
        u_hbm, src_hbm, dst_hbm, zr_hbm, out_hbm,
        src_v, dst_v, rows_v, acc_sh,
    ):
        cid = lax.axis_index("c")
        sid = lax.axis_index("s")
        wid = sid * NC + cid
        pltpu.sync_copy(src_hbm.at[wid], src_v)
        pltpu.sync_copy(dst_hbm.at[wid], dst_v)
        pltpu.sync_copy(zr_hbm, acc_sh.at[pl.ds(sid * RPT, RPT)])
        plsc.subcore_barrier()

        def body(j, _):
            pltpu.sync_copy(u_hbm.at[src_v.at[j]], rows_v)
            pltpu.sync_copy(rows_v, acc_sh.at[dst_v.at[j]], add=True)
            return 0

        lax.fori_loop(0, CH, body, 0)
        plsc.subcore_barrier()
        pltpu.sync_copy(
            acc_sh.at[pl.ds(sid * RPT, RPT)],
            out_hbm.at[cid, pl.ds(sid * RPT, RPT)],
        )

    return scat_k(u, srcA, dstA, zrows)


def _const_spec(shape):
    return pl.BlockSpec(shape, lambda i: tuple(0 for _ in shape))


def _pre_call(x_p, We1, be1, We2, be2, Wg1, degp_t):
    def body(x_ref, w1, b1, w2, b2, wg, dg, u_ref, dv_ref):
        xb = x_ref[...]
        h = jnp.maximum(
            jnp.dot(xb, w1[...], preferred_element_type=jnp.float32) + b1[...], 0.0
        )
        h0 = jnp.dot(h, w2[...], preferred_element_type=jnp.float32) + b2[...]
        deg = jnp.sum(dg[...], axis=1, keepdims=True) + 1.0
        dv = lax.rsqrt(deg)
        u_ref[...] = dv * jnp.dot(h0, wg[...], preferred_element_type=jnp.float32)
        dv_ref[...] = dv

    return pl.pallas_call(
        body,
        grid=(GG,),
        in_specs=[
            pl.BlockSpec((RB, HH), lambda i: (i, 0)),
            _const_spec((HH, HH)),
            _const_spec((1, HH)),
            _const_spec((HH, HH)),
            _const_spec((1, HH)),
            _const_spec((HH, HH)),
            pl.BlockSpec((RB, NC), lambda i: (i, 0)),
        ],
        out_specs=[
            pl.BlockSpec((RB, HH), lambda i: (i, 0)),
            pl.BlockSpec((RB, 1), lambda i: (i, 0)),
        ],
        out_shape=[
            jax.ShapeDtypeStruct((NP, HH), jnp.float32),
            jax.ShapeDtypeStruct((NP, 1), jnp.float32),
        ],
    )(x_p, We1, be1, We2, be2, Wg1, degp_t)


def _mid_call(sp, u, dinv, bg, wgn):
    def body(s_ref, u_ref, dv_ref, b_ref, w_ref, un_ref):
        s = s_ref[0] + s_ref[1]
        dv = dv_ref[...]
        hh = jnp.maximum(dv * (s + u_ref[...]) + b_ref[...], 0.0)
        un_ref[...] = dv * jnp.dot(hh, w_ref[...], preferred_element_type=jnp.float32)

    return pl.pallas_call(
        body,
        grid=(GG,),
        in_specs=[
            pl.BlockSpec((NC, RB, HH), lambda i: (0, i, 0)),
            pl.BlockSpec((RB, HH), lambda i: (i, 0)),
            pl.BlockSpec((RB, 1), lambda i: (i, 0)),
            _const_spec((1, HH)),
            _const_spec((HH, HH)),
        ],
        out_specs=pl.BlockSpec((RB, HH), lambda i: (i, 0)),
        out_shape=jax.ShapeDtypeStruct((NP, HH), jnp.float32),
    )(sp, u, dinv, bg, wgn)


def _post_call(sp, u, dinv, bg3, Wd1, bd1, Wd2, bd2, Wp1, bp1, Wp2, bp2):
    def body(
        s_ref, u_ref, dv_ref, b_ref, wd1, bd1r, wd2, bd2r, wp1, bp1r, wp2, bp2r,
        ge_ref, h_ref, rec_ref, sum_sc, max_sc,
    ):
        i = pl.program_id(0)
        s = s_ref[0] + s_ref[1]
        dv = dv_ref[...]
        hh = jnp.maximum(dv * (s + u_ref[...]) + b_ref[...], 0.0)
        h_ref[...] = hh
        r1 = jnp.maximum(
            jnp.dot(hh, wd1[...], preferred_element_type=jnp.float32) + bd1r[...], 0.0
        )
        rec_ref[...] = (
            jnp.dot(r1, wd2[...], preferred_element_type=jnp.float32) + bd2r[...]
        )
        rows = i * RB + lax.broadcasted_iota(jnp.int32, (RB, 1), 0)
        msk = rows < NN
        ps = jnp.sum(jnp.where(msk, hh, 0.0), axis=0, keepdims=True)
        px = jnp.max(jnp.where(msk, hh, -jnp.inf), axis=0, keepdims=True)

        @pl.when(i == 0)
        def _():
            sum_sc[...] = ps
            max_sc[...] = px

        @pl.when(i > 0)
        def _():
            sum_sc[...] += ps
            max_sc[...] = jnp.maximum(max_sc[...], px)

        @pl.when(i == GG - 1)
        def _():
            g = jnp.concatenate([sum_sc[...] * (1.0 / NN), max_sc[...]], axis=1)
            gh = jnp.maximum(
                jnp.dot(g, wp1[...], preferred_element_type=jnp.float32) + bp1r[...],
                0.0,
            )
            ge_ref[...] = (
                jnp.dot(gh, wp2[...], preferred_element_type=jnp.float32) + bp2r[...]
            )

    return pl.pallas_call(
        body,
        grid=(GG,),
        in_specs=[
            pl.BlockSpec((NC, RB, HH), lambda i: (0, i, 0)),
            pl.BlockSpec((RB, HH), lambda i: (i, 0)),
            pl.BlockSpec((RB, 1), lambda i: (i, 0)),
            _const_spec((1, HH)),
            _const_spec((HH, HH)),
            _const_spec((1, HH)),
            _const_spec((HH, HH)),
            _const_spec((1, HH)),
            _const_spec((2 * HH, HH)),
            _const_spec((1, HH)),
            _const_spec((HH, HH)),
            _const_spec((1, HH)),
        ],
        out_specs=[
            _const_spec((1, HH)),
            pl.BlockSpec((RB, HH), lambda i: (i, 0)),
            pl.BlockSpec((RB, HH), lambda i: (i, 0)),
        ],
        out_shape=[
            jax.ShapeDtypeStruct((1, HH), jnp.float32),
            jax.ShapeDtypeStruct((NP, HH), jnp.float32),
            jax.ShapeDtypeStruct((NP, HH), jnp.float32),
        ],
        scratch_shapes=[
            pltpu.VMEM((1, HH), jnp.float32),
            pltpu.VMEM((1, HH), jnp.float32),
        ],
    )(sp, u, dinv, bg3, Wd1, bd1, Wd2, bd2, Wp1, bp1, Wp2, bp2)


def kernel(
    x, We1, be1, We2, be2, Wg1, bg1, Wg2, bg2, Wg3, bg3,
    Wp1, bp1, Wp2, bp2, Wd1, bd1, Wd2, bd2, edge_index,
):
    f32 = jnp.float32
    src = edge_index[0]
    dst = edge_index[1]
    pad = EP - EE
    src_p = jnp.concatenate([src, jnp.zeros((pad,), src.dtype)])
    pad_dst = NN + (jnp.arange(pad, dtype=dst.dtype) % (NP - NN))
    dst_p = jnp.concatenate([dst, pad_dst])
    dst3 = dst_p.reshape(NW, CH, KE)

    srcA = src_p.reshape(NW, CH, KE)
    dstA = dst_p.reshape(NW, CH, KE)
    x_p = jnp.pad(x, ((0, NP - NN), (0, 0)))
    zcol = jnp.zeros((RPT,), f32)
    zrows = jnp.zeros((RPT, HH), f32)

    degp = _deg_call(dst3, zcol)
    u1, dinv = _pre_call(
        x_p, We1, be1.reshape(1, HH), We2, be2.reshape(1, HH), Wg1, degp.T
    )
    s1 = _scatter_call(u1, srcA, dstA, zrows)
    u2 = _mid_call(s1, u1, dinv, bg1.reshape(1, HH), Wg2)
    s2 = _scatter_call(u2, srcA, dstA, zrows)
    u3 = _mid_call(s2, u2, dinv, bg2.reshape(1, HH), Wg3)
    s3 = _scatter_call(u3, srcA, dstA, zrows)
    ge, h3, rec = _post_call(
        s3, u3, dinv, bg3.reshape(1, HH),
        Wd1, bd1.reshape(1, HH), Wd2, bd2.reshape(1, HH),
        Wp1, bp1.reshape(1, HH), Wp2, bp2.reshape(1, HH),
    )
    return ge, h3[:NN], rec[:NN]
